# Initial kernel scaffold; baseline (speedup 1.0000x reference)
import jax
import jax.numpy as jnp
from jax.experimental import pallas as pl


def kernel(x, edge_index, edge_weight, ln_gamma, ln_beta, W, b):
    raise NotImplementedError("write your pallas kernel here")


# speedup vs baseline: 8.8106x; 8.8106x over previous
import functools

import jax
import jax.numpy as jnp
from jax import lax
from jax.experimental import pallas as pl
from jax.experimental.pallas import tpu as pltpu
from jax.experimental.pallas import tpu_sc as plsc

N = 10000
NP = 10240
E = 320000
EC = 157
CB = 128
EP = 16 * EC * CB
F = 128
FH = 64
NCLS = 40
ND = 640
NSUB = 16
NCORE = 2

_i32 = jnp.int32
_f32 = jnp.float32


def _ln_body(x_ref, g_ref, b_ref, o_ref):
    xv = x_ref[...]
    mean = jnp.mean(xv, axis=1, keepdims=True)
    xc = xv - mean
    var = jnp.mean(xc * xc, axis=1, keepdims=True)
    hn = xc * lax.rsqrt(var + 1e-5) * g_ref[...] + b_ref[...]
    o_ref[0] = hn[:, :FH]
    o_ref[1] = hn[:, FH:]


def _layernorm_split(xp, gamma, beta):
    BN = 2048
    return pl.pallas_call(
        _ln_body,
        grid=(NP // BN,),
        in_specs=[
            pl.BlockSpec((BN, F), lambda i: (i, 0)),
            pl.BlockSpec((F,), lambda i: (0,)),
            pl.BlockSpec((F,), lambda i: (0,)),
        ],
        out_specs=pl.BlockSpec((2, BN, FH), lambda i: (0, i, 0)),
        out_shape=jax.ShapeDtypeStruct((2, NP, FH), _f32),
    )(xp, gamma, beta)


def _proj_body(h_ref, w_ref, b_ref, o_ref):
    hcat = jnp.concatenate([h_ref[0], h_ref[1]], axis=1)
    logits = jnp.dot(hcat, w_ref[...].T, preferred_element_type=_f32) + b_ref[...]
    m = jnp.max(logits, axis=1, keepdims=True)
    ex = jnp.exp(logits - m)
    lse = jnp.log(jnp.sum(ex, axis=1, keepdims=True)) + m
    o_ref[...] = logits - lse


def _project(h3, W, b):
    BN = 2048
    return pl.pallas_call(
        _proj_body,
        grid=(NP // BN,),
        in_specs=[
            pl.BlockSpec((2, BN, FH), lambda i: (0, i, 0)),
            pl.BlockSpec((NCLS, F), lambda i: (0, 0)),
            pl.BlockSpec((NCLS,), lambda i: (0,)),
        ],
        out_specs=pl.BlockSpec((BN, NCLS), lambda i: (i, 0)),
        out_shape=jax.ShapeDtypeStruct((NP, NCLS), _f32),
    )(h3, W, b)


def _full16(v):
    return jnp.full((16,), v, _i32)


def _newton_rsqrt(d):
    ii = plsc.bitcast(d, _i32)
    mg = jnp.int32(0x5F3759DF) - (ii >> 1)
    x = plsc.bitcast(mg, _f32)
    for _ in range(3):
        x = x * (1.5 - 0.5 * d * x * x)
    return x


def _degnorm_body(row2, col3, ew2, norm3, s_hbm,
                  rowb, colb, ewb, valb, degb, dslice, sqb, dinvb, ntmp,
                  deg_sp, dinv_sp):
    c = lax.axis_index("c")
    s = lax.axis_index("s")
    nb = ND * s

    def _z(i, _):
        degb[i, :] = jnp.zeros((16,), _f32)
        return 0
    lax.fori_loop(0, ND, _z, 0)
    pltpu.sync_copy(degb, deg_sp.at[pl.ds(nb, ND)])
    plsc.subcore_barrier()

    pltpu.sync_copy(row2.at[s], rowb)
    pltpu.sync_copy(col3.at[s], colb)
    pltpu.sync_copy(ew2.at[s], ewb)

    def _chunk(j, _):
        def _edge(e, _):
            bc = plsc.load_gather(ewb, [_full16(j * CB + e)])
            valb[e, :] = bc
            return 0
        lax.fori_loop(0, CB, _edge, 0)
        pltpu.sync_copy(valb, deg_sp.at[colb.at[j]], add=True)
        return 0
    lax.fori_loop(0, EC, _chunk, 0)
    plsc.subcore_barrier()

    pltpu.sync_copy(deg_sp.at[pl.ds(nb, ND)], degb)
    lanes = lax.iota(_i32, 16)

    def _dv(k, _):
        r0 = 16 * k
        dv = plsc.load_gather(degb, [lanes + r0, lanes])
        x = _newton_rsqrt(dv + 1.0)
        dslice[pl.ds(r0, 16)] = x
        sqb[pl.ds(r0, 16)] = x * x
        return 0
    lax.fori_loop(0, ND // 16, _dv, 0)
    pltpu.sync_copy(dslice, dinv_sp.at[pl.ds(nb, ND)])

    @pl.when(c == 0)
    def _():
        pltpu.sync_copy(sqb, s_hbm.at[pl.ds(nb, ND)])

    plsc.subcore_barrier()
    pltpu.sync_copy(dinv_sp, dinvb)

    def _nchunk(jj, _):
        j = 2 * jj + c

        @pl.when(j < EC)
        def _():
            def _grp(m, _):
                off = 16 * m
                rv = rowb[pl.ds(j * CB + off, 16)]
                cv = colb[j, pl.ds(off, 16)]
                ev = ewb[pl.ds(j * CB + off, 16)]
                dr = plsc.load_gather(dinvb, [rv])
                dc = plsc.load_gather(dinvb, [cv])
                ntmp[pl.ds(off, 16)] = dr * ev * dc
                return 0
            lax.fori_loop(0, CB // 16, _grp, 0)
            pltpu.sync_copy(ntmp, norm3.at[s, j])
        return 0
    lax.fori_loop(0, (EC + 1) // 2, _nchunk, 0)


def _degnorm(row2, col3, ew2):
    mesh = plsc.VectorSubcoreMesh(core_axis_name="c", subcore_axis_name="s")
    f = pl.kernel(
        _degnorm_body,
        out_type=(
            jax.ShapeDtypeStruct((NSUB, EC, CB), _f32),
            jax.ShapeDtypeStruct((NP,), _f32),
        ),
        mesh=mesh,
        compiler_params=pltpu.CompilerParams(
            needs_layout_passes=False, use_tc_tiling_on_sc=False),
        scratch_types=[
            pltpu.VMEM((EC * CB,), _i32),
            pltpu.VMEM((EC, CB), _i32),
            pltpu.VMEM((EC * CB,), _f32),
            pltpu.VMEM((CB, 16), _f32),
            pltpu.VMEM((ND, 16), _f32),
            pltpu.VMEM((ND,), _f32),
            pltpu.VMEM((ND,), _f32),
            pltpu.VMEM((NP,), _f32),
            pltpu.VMEM((CB,), _f32),
            pltpu.VMEM_SHARED((NP, 16), _f32),
            pltpu.VMEM_SHARED((NP,), _f32),
        ],
    )
    return f(row2, col3, ew2)


def _prop_body(hf, row2, col3, norm2, s_hbm, of,
               rowb, colb, normb, db, sv, accum):
    c = lax.axis_index("c")
    s = lax.axis_index("s")
    nb = ND * s
    hoff = c * NP

    for j in range(ND // CB):
        pltpu.sync_copy(hf.at[pl.ds(hoff + nb + CB * j, CB)], db)
        pltpu.sync_copy(s_hbm.at[pl.ds(nb + CB * j, CB)], sv)

        def _row(e, _):
            bc = plsc.load_gather(sv, [_full16(e)]).astype(_f32)
            for k in range(FH // 16):
                db[e, pl.ds(16 * k, 16)] = db[e, pl.ds(16 * k, 16)] * bc
            return 0
        lax.fori_loop(0, CB, _row, 0)
        pltpu.sync_copy(db, accum.at[pl.ds(nb + CB * j, CB)])
    plsc.subcore_barrier()

    pltpu.sync_copy(row2.at[s], rowb)
    pltpu.sync_copy(col3.at[s], colb)
    pltpu.sync_copy(norm2.at[s], normb)

    def _shift(m, _):
        off = 16 * m
        rowb[pl.ds(off, 16)] = rowb[pl.ds(off, 16)] + hoff
        return 0
    lax.fori_loop(0, EC * CB // 16, _shift, 0)

    def _chunk(j, _):
        pltpu.sync_copy(hf.at[rowb.at[pl.ds(j * CB, CB)]], db)

        def _edge(e, _):
            bc = plsc.load_gather(normb, [_full16(j * CB + e)])
            for k in range(FH // 16):
                db[e, pl.ds(16 * k, 16)] = db[e, pl.ds(16 * k, 16)] * bc
            return 0
        lax.fori_loop(0, CB, _edge, 0)
        pltpu.sync_copy(db, accum.at[colb.at[j]], add=True)
        return 0
    lax.fori_loop(0, EC, _chunk, 0)
    plsc.subcore_barrier()

    for j in range(ND // CB):
        pltpu.sync_copy(accum.at[pl.ds(nb + CB * j, CB)], db)
        pltpu.sync_copy(db, of.at[pl.ds(hoff + nb + CB * j, CB)])


def _propagate(hf, row2, col3, norm2, s_hbm):
    mesh = plsc.VectorSubcoreMesh(core_axis_name="c", subcore_axis_name="s")
    f = pl.kernel(
        _prop_body,
        out_type=jax.ShapeDtypeStruct((2 * NP, FH), _f32),
        mesh=mesh,
        compiler_params=pltpu.CompilerParams(
            needs_layout_passes=False, use_tc_tiling_on_sc=False),
        scratch_types=[
            pltpu.VMEM((EC * CB,), _i32),
            pltpu.VMEM((EC, CB), _i32),
            pltpu.VMEM((EC * CB,), _f32),
            pltpu.VMEM((CB, FH), _f32),
            pltpu.VMEM((CB,), _f32),
            pltpu.VMEM_SHARED((NP, FH), _f32),
        ],
    )
    return f(hf, row2, col3, norm2, s_hbm)


def kernel(x, edge_index, edge_weight, ln_gamma, ln_beta, W, b):
    row = edge_index[0]
    col = edge_index[1]
    pad_e = EP - E
    row2 = jnp.pad(row, (0, pad_e)).reshape(NSUB, EC * CB)
    col3 = jnp.pad(col, (0, pad_e)).reshape(NSUB, EC, CB)
    ew2 = jnp.pad(edge_weight, (0, pad_e)).reshape(NSUB, EC * CB)
    xp = jnp.pad(x, ((0, NP - N), (0, 0)))

    h3 = _layernorm_split(xp, ln_gamma, ln_beta)
    norm3, s_hbm = _degnorm(row2, col3, ew2)
    norm2 = norm3.reshape(NSUB, EC * CB)

    hf = h3.reshape(2 * NP, FH)
    for _ in range(2):
        hf = _propagate(hf, row2, col3, norm2, s_hbm)

    out = _project(hf.reshape(2, NP, FH), W, b)
    return out[:N]


# speedup vs baseline: 20.5127x; 2.3282x over previous
import functools

import jax
import jax.numpy as jnp
from jax import lax
from jax.experimental import pallas as pl
from jax.experimental.pallas import tpu as pltpu
from jax.experimental.pallas import tpu_sc as plsc

N = 10000
NP = 10240
E = 320000
EC = 157
CB = 128
EP = 16 * EC * CB
F = 128
FH = 64
NCLS = 40
ND = 640
NSUB = 16
JSPLIT = 79

_i32 = jnp.int32
_f32 = jnp.float32

_SC_PARAMS = pltpu.CompilerParams(
    needs_layout_passes=False, use_tc_tiling_on_sc=False)


def _ln_body(x_ref, g_ref, b_ref, o_ref):
    xv = x_ref[...]
    mean = jnp.mean(xv, axis=1, keepdims=True)
    xc = xv - mean
    var = jnp.mean(xc * xc, axis=1, keepdims=True)
    hn = xc * lax.rsqrt(var + 1e-5) * g_ref[...] + b_ref[...]
    o_ref[0] = hn[:, :FH]
    o_ref[1] = hn[:, FH:]


def _layernorm_split(xp, gamma, beta):
    BN = 2048
    return pl.pallas_call(
        _ln_body,
        grid=(NP // BN,),
        in_specs=[
            pl.BlockSpec((BN, F), lambda i: (i, 0)),
            pl.BlockSpec((F,), lambda i: (0,)),
            pl.BlockSpec((F,), lambda i: (0,)),
        ],
        out_specs=pl.BlockSpec((2, BN, FH), lambda i: (0, i, 0)),
        out_shape=jax.ShapeDtypeStruct((2, NP, FH), _f32),
    )(xp, gamma, beta)


def _proj_body(h_ref, w_ref, b_ref, o_ref):
    hcat = jnp.concatenate([h_ref[0], h_ref[1]], axis=1)
    logits = jnp.dot(hcat, w_ref[...].T, preferred_element_type=_f32) + b_ref[...]
    m = jnp.max(logits, axis=1, keepdims=True)
    ex = jnp.exp(logits - m)
    lse = jnp.log(jnp.sum(ex, axis=1, keepdims=True)) + m
    o_ref[...] = logits - lse


def _project(h3, W, b):
    BN = 2048
    return pl.pallas_call(
        _proj_body,
        grid=(NP // BN,),
        in_specs=[
            pl.BlockSpec((2, BN, FH), lambda i: (0, i, 0)),
            pl.BlockSpec((NCLS, F), lambda i: (0, 0)),
            pl.BlockSpec((NCLS,), lambda i: (0,)),
        ],
        out_specs=pl.BlockSpec((BN, NCLS), lambda i: (i, 0)),
        out_shape=jax.ShapeDtypeStruct((NP, NCLS), _f32),
    )(h3, W, b)


def _full16(v):
    return jnp.full((16,), v, _i32)


def _newton_rsqrt(d):
    ii = plsc.bitcast(d, _i32)
    mg = jnp.int32(0x5F3759DF) - (ii >> 1)
    x = plsc.bitcast(mg, _f32)
    for _ in range(3):
        x = x * (1.5 - 0.5 * d * x * x)
    return x


def _degnorm_body(row2, col3, ew2, norm3, s_hbm,
                  rowb, colb, ewb, valb0, valb1, degb, dslice, sqb, dinvb,
                  normout, sem0, sem1, deg_sp, dinv_sp):
    c = lax.axis_index("c")
    s = lax.axis_index("s")
    nb = ND * s
    valbs = (valb0, valb1)
    sems = (sem0, sem1)

    @plsc.parallel_loop(0, ND, unroll=8)
    def _(i):
        degb[i, :] = jnp.zeros((16,), _f32)
    pltpu.sync_copy(degb, deg_sp.at[pl.ds(nb, ND)])
    plsc.subcore_barrier()

    pltpu.sync_copy(row2.at[s], rowb)
    pltpu.sync_copy(col3.at[s], colb)
    pltpu.sync_copy(ew2.at[s], ewb)

    def _build(j, u):
        @plsc.parallel_loop(0, CB, unroll=8)
        def _(e):
            bc = plsc.load_gather(ewb, [_full16(j * CB + e)])
            valbs[u][e, :] = bc

    def _dround(r, _):
        for u in range(2):
            j = 2 * r + u

            @pl.when(j >= 2)
            def _():
                pltpu.make_async_copy(
                    valbs[u], deg_sp.at[colb.at[j - 2]], sems[u]).wait()
            _build(j, u)
            pltpu.async_copy(valbs[u], deg_sp.at[colb.at[j]], sems[u], add=True)
        return 0
    lax.fori_loop(0, EC // 2, _dround, 0)
    pltpu.make_async_copy(valbs[0], deg_sp.at[colb.at[EC - 3]], sems[0]).wait()
    _build(EC - 1, 0)
    pltpu.async_copy(valbs[0], deg_sp.at[colb.at[EC - 1]], sems[0], add=True)
    pltpu.make_async_copy(valbs[1], deg_sp.at[colb.at[EC - 2]], sems[1]).wait()
    pltpu.make_async_copy(valbs[0], deg_sp.at[colb.at[EC - 1]], sems[0]).wait()
    plsc.subcore_barrier()

    pltpu.sync_copy(deg_sp.at[pl.ds(nb, ND)], degb)
    lanes = lax.iota(_i32, 16)

    @plsc.parallel_loop(0, ND // 16, unroll=4)
    def _(k):
        r0 = 16 * k
        dv = plsc.load_gather(degb, [lanes + r0, lanes])
        x = _newton_rsqrt(dv + 1.0)
        dslice[pl.ds(r0, 16)] = x
        sqb[pl.ds(r0, 16)] = x * x
    pltpu.sync_copy(dslice, dinv_sp.at[pl.ds(nb, ND)])

    @pl.when(c == 0)
    def _():
        pltpu.sync_copy(sqb, s_hbm.at[pl.ds(nb, ND)])

    plsc.subcore_barrier()
    pltpu.sync_copy(dinv_sp, dinvb)

    def _norm_range(jlo, jhi):
        def _nj(j, _):
            @plsc.parallel_loop(0, CB // 16, unroll=4)
            def _(m):
                off = j * CB + 16 * m
                rv = rowb[pl.ds(off, 16)]
                cv = colb[j, pl.ds(16 * m, 16)]
                ev = ewb[pl.ds(off, 16)]
                dr = plsc.load_gather(dinvb, [rv])
                dc = plsc.load_gather(dinvb, [cv])
                normout[j - jlo, pl.ds(16 * m, 16)] = dr * ev * dc
            return 0
        lax.fori_loop(jlo, jhi, _nj, 0)
        pltpu.sync_copy(normout.at[pl.ds(0, jhi - jlo)],
                        norm3.at[s, pl.ds(jlo, jhi - jlo)])

    @pl.when(c == 0)
    def _():
        _norm_range(0, JSPLIT)

    @pl.when(c == 1)
    def _():
        _norm_range(JSPLIT, EC)


def _degnorm(row2, col3, ew2):
    mesh = plsc.VectorSubcoreMesh(core_axis_name="c", subcore_axis_name="s")
    f = pl.kernel(
        _degnorm_body,
        out_type=(
            jax.ShapeDtypeStruct((NSUB, EC, CB), _f32),
            jax.ShapeDtypeStruct((NP,), _f32),
        ),
        mesh=mesh,
        compiler_params=_SC_PARAMS,
        scratch_types=[
            pltpu.VMEM((EC * CB,), _i32),
            pltpu.VMEM((EC, CB), _i32),
            pltpu.VMEM((EC * CB,), _f32),
            pltpu.VMEM((CB, 16), _f32),
            pltpu.VMEM((CB, 16), _f32),
            pltpu.VMEM((ND, 16), _f32),
            pltpu.VMEM((ND,), _f32),
            pltpu.VMEM((ND,), _f32),
            pltpu.VMEM((NP,), _f32),
            pltpu.VMEM((JSPLIT, CB), _f32),
            pltpu.SemaphoreType.DMA,
            pltpu.SemaphoreType.DMA,
            pltpu.VMEM_SHARED((NP, 16), _f32),
            pltpu.VMEM_SHARED((NP,), _f32),
        ],
    )
    return f(row2, col3, ew2)


def _prop_body(hf, row2, col3, norm2, s_hbm, of,
               rowb, colb, normb, gb0, gb1, gb2, sv,
               gs0, gs1, gs2, accum):
    c = lax.axis_index("c")
    s = lax.axis_index("s")
    nb = ND * s
    hoff = c * NP
    gbs = (gb0, gb1, gb2)
    gss = (gs0, gs1, gs2)

    for j in range(ND // CB):
        pltpu.sync_copy(hf.at[pl.ds(hoff + nb + CB * j, CB)], gb0)
        pltpu.sync_copy(s_hbm.at[pl.ds(nb + CB * j, CB)], sv)

        @plsc.parallel_loop(0, CB, unroll=8)
        def _(e):
            bc = plsc.load_gather(sv, [_full16(e)])
            for k in range(FH // 16):
                gb0[e, pl.ds(16 * k, 16)] = gb0[e, pl.ds(16 * k, 16)] * bc
        pltpu.sync_copy(gb0, accum.at[pl.ds(nb + CB * j, CB)])
    plsc.subcore_barrier()

    pltpu.sync_copy(row2.at[s], rowb)
    pltpu.sync_copy(col3.at[s], colb)
    pltpu.sync_copy(norm2.at[s], normb)

    @plsc.parallel_loop(0, EC * CB // 16, unroll=8)
    def _(m):
        off = 16 * m
        rowb[pl.ds(off, 16)] = rowb[pl.ds(off, 16)] + hoff

    def _work(j, u, fire):
        pltpu.make_async_copy(
            hf.at[rowb.at[pl.ds(j * CB, CB)]], gbs[u], gss[u]).wait()

        @plsc.parallel_loop(0, CB, unroll=4)
        def _(e):
            bc = plsc.load_gather(normb, [_full16(j * CB + e)])
            for k in range(FH // 16):
                gbs[u][e, pl.ds(16 * k, 16)] = gbs[u][e, pl.ds(16 * k, 16)] * bc
        pltpu.sync_copy(gbs[u], accum.at[colb.at[j]], add=True)
        if fire:
            jn = j + 3

            @pl.when(jn < EC)
            def _():
                pltpu.async_copy(
                    hf.at[rowb.at[pl.ds(jn * CB, CB)]], gbs[u], gss[u])

    for u in range(3):
        pltpu.async_copy(hf.at[rowb.at[pl.ds(u * CB, CB)]], gbs[u], gss[u])

    def _round(r, _):
        for u in range(3):
            _work(3 * r + u, u, True)
        return 0
    lax.fori_loop(0, EC // 3, _round, 0)
    _work(EC - 1, 0, False)
    plsc.subcore_barrier()

    pltpu.sync_copy(accum.at[pl.ds(nb, ND)], of.at[pl.ds(hoff + nb, ND)])


def _propagate(hf, row2, col3, norm2, s_hbm):
    mesh = plsc.VectorSubcoreMesh(core_axis_name="c", subcore_axis_name="s")
    f = pl.kernel(
        _prop_body,
        out_type=jax.ShapeDtypeStruct((2 * NP, FH), _f32),
        mesh=mesh,
        compiler_params=_SC_PARAMS,
        scratch_types=[
            pltpu.VMEM((EC * CB,), _i32),
            pltpu.VMEM((EC, CB), _i32),
            pltpu.VMEM((EC * CB,), _f32),
            pltpu.VMEM((CB, FH), _f32),
            pltpu.VMEM((CB, FH), _f32),
            pltpu.VMEM((CB, FH), _f32),
            pltpu.VMEM((CB,), _f32),
            pltpu.SemaphoreType.DMA,
            pltpu.SemaphoreType.DMA,
            pltpu.SemaphoreType.DMA,
            pltpu.VMEM_SHARED((NP, FH), _f32),
        ],
    )
    return f(hf, row2, col3, norm2, s_hbm)


def kernel(x, edge_index, edge_weight, ln_gamma, ln_beta, W, b):
    row = edge_index[0]
    col = edge_index[1]
    pad_e = EP - E
    row2 = jnp.pad(row, (0, pad_e)).reshape(NSUB, EC * CB)
    col3 = jnp.pad(col, (0, pad_e)).reshape(NSUB, EC, CB)
    ew2 = jnp.pad(edge_weight, (0, pad_e)).reshape(NSUB, EC * CB)
    xp = jnp.pad(x, ((0, NP - N), (0, 0)))

    h3 = _layernorm_split(xp, ln_gamma, ln_beta)
    norm3, s_hbm = _degnorm(row2, col3, ew2)
    norm2 = norm3.reshape(NSUB, EC * CB)

    hf = h3.reshape(2 * NP, FH)
    for _ in range(2):
        hf = _propagate(hf, row2, col3, norm2, s_hbm)

    out = _project(hf.reshape(2, NP, FH), W, b)
    return out[:N]


# speedup vs baseline: 24.2237x; 1.1809x over previous
import functools

import jax
import jax.numpy as jnp
from jax import lax
from jax.experimental import pallas as pl
from jax.experimental.pallas import tpu as pltpu
from jax.experimental.pallas import tpu_sc as plsc

N = 10000
NP = 10240
E = 320000
EC = 157
CB = 128
EP = 16 * EC * CB
F = 128
FH = 64
NCLS = 40
ND = 640
NSUB = 16

_i32 = jnp.int32
_f32 = jnp.float32

_SC_PARAMS = pltpu.CompilerParams(
    needs_layout_passes=False, use_tc_tiling_on_sc=False)


def _ln_body(x_ref, g_ref, b_ref, o_ref):
    i = pl.program_id(0)
    xv = x_ref[...]
    mean = jnp.mean(xv, axis=1, keepdims=True)
    xc = xv - mean
    var = jnp.mean(xc * xc, axis=1, keepdims=True)
    hn = xc * lax.rsqrt(var + 1e-5) * g_ref[...] + b_ref[...]
    ids = lax.broadcasted_iota(_i32, hn.shape, 0) + i * hn.shape[0]
    hn = jnp.where(ids < N, hn, 0.0)
    o_ref[0] = hn[:, :FH]
    o_ref[1] = hn[:, FH:]


def _layernorm_split(xp, gamma, beta):
    BN = 2048
    return pl.pallas_call(
        _ln_body,
        grid=(NP // BN,),
        in_specs=[
            pl.BlockSpec((BN, F), lambda i: (i, 0)),
            pl.BlockSpec((F,), lambda i: (0,)),
            pl.BlockSpec((F,), lambda i: (0,)),
        ],
        out_specs=pl.BlockSpec((2, BN, FH), lambda i: (0, i, 0)),
        out_shape=jax.ShapeDtypeStruct((2, NP, FH), _f32),
    )(xp, gamma, beta)


def _proj_body(h_ref, w_ref, b_ref, o_ref):
    hcat = jnp.concatenate([h_ref[0], h_ref[1]], axis=1)
    logits = jnp.dot(hcat, w_ref[...].T, preferred_element_type=_f32) + b_ref[...]
    m = jnp.max(logits, axis=1, keepdims=True)
    ex = jnp.exp(logits - m)
    lse = jnp.log(jnp.sum(ex, axis=1, keepdims=True)) + m
    o_ref[...] = logits - lse


def _project(h3, W, b):
    BN = 2048
    return pl.pallas_call(
        _proj_body,
        grid=(NP // BN,),
        in_specs=[
            pl.BlockSpec((2, BN, FH), lambda i: (0, i, 0)),
            pl.BlockSpec((NCLS, F), lambda i: (0, 0)),
            pl.BlockSpec((NCLS,), lambda i: (0,)),
        ],
        out_specs=pl.BlockSpec((BN, NCLS), lambda i: (i, 0)),
        out_shape=jax.ShapeDtypeStruct((NP, NCLS), _f32),
    )(h3, W, b)


def _full16(v):
    return jnp.full((16,), v, _i32)


def _newton_rsqrt(d):
    ii = plsc.bitcast(d, _i32)
    mg = jnp.int32(0x5F3759DF) - (ii >> 1)
    x = plsc.bitcast(mg, _f32)
    for _ in range(3):
        x = x * (1.5 - 0.5 * d * x * x)
    return x


def _deg_body(col3, ew2, dinv_hbm, sq_hbm,
              colb, ewb, valb0, valb1, dslice, sqb,
              vs0, vs1, deg_sp):
    c = lax.axis_index("c")
    s = lax.axis_index("s")
    nb = ND * s
    valbs = (valb0, valb1)
    vss = (vs0, vs1)
    lanes = lax.iota(_i32, 16)

    pltpu.sync_copy(col3.at[s], colb)
    pltpu.sync_copy(ew2.at[s], ewb)

    @plsc.parallel_loop(0, CB, unroll=8)
    def _(i):
        valb0[i, :] = jnp.zeros((16,), _f32)
    for j in range(ND // CB):
        pltpu.sync_copy(valb0, deg_sp.at[pl.ds(nb + CB * j, CB)])
    plsc.subcore_barrier()

    def _build(j, u):
        @plsc.parallel_loop(0, CB, unroll=8)
        def _(e):
            bc = plsc.load_gather(ewb, [_full16(j * CB + e)])
            valbs[u][e, :] = bc

    def _dround(r, _):
        for u in range(2):
            j = 2 * r + u

            @pl.when(j >= 2)
            def _():
                pltpu.make_async_copy(
                    valbs[u], deg_sp.at[colb.at[j - 2]], vss[u]).wait()
            _build(j, u)
            pltpu.async_copy(valbs[u], deg_sp.at[colb.at[j]], vss[u], add=True)
        return 0
    lax.fori_loop(0, EC // 2, _dround, 0)
    pltpu.make_async_copy(valbs[0], deg_sp.at[colb.at[EC - 3]], vss[0]).wait()
    _build(EC - 1, 0)
    pltpu.async_copy(valbs[0], deg_sp.at[colb.at[EC - 1]], vss[0], add=True)
    pltpu.make_async_copy(valbs[1], deg_sp.at[colb.at[EC - 2]], vss[1]).wait()
    pltpu.make_async_copy(valbs[0], deg_sp.at[colb.at[EC - 1]], vss[0]).wait()
    plsc.subcore_barrier()

    for jj in range(ND // CB):
        pltpu.sync_copy(deg_sp.at[pl.ds(nb + CB * jj, CB)], valb0)

        @plsc.parallel_loop(0, CB // 16, unroll=4)
        def _(k):
            dv = plsc.load_gather(valb0, [lanes + 16 * k, lanes])
            x = _newton_rsqrt(dv + 1.0)
            dslice[pl.ds(CB * jj + 16 * k, 16)] = x
            sqb[pl.ds(CB * jj + 16 * k, 16)] = x * x

    @pl.when(c == 0)
    def _():
        pltpu.sync_copy(dslice, dinv_hbm.at[pl.ds(nb, ND)])
        pltpu.sync_copy(sqb, sq_hbm.at[pl.ds(nb, ND)])


def _deg(col3, ew2):
    mesh = plsc.VectorSubcoreMesh(core_axis_name="c", subcore_axis_name="s")
    f = pl.kernel(
        _deg_body,
        out_type=(
            jax.ShapeDtypeStruct((NP,), _f32),
            jax.ShapeDtypeStruct((NP,), _f32),
        ),
        mesh=mesh,
        compiler_params=_SC_PARAMS,
        scratch_types=[
            pltpu.VMEM((EC, CB), _i32),
            pltpu.VMEM((EC * CB,), _f32),
            pltpu.VMEM((CB, 16), _f32),
            pltpu.VMEM((CB, 16), _f32),
            pltpu.VMEM((ND,), _f32),
            pltpu.VMEM((ND,), _f32),
            pltpu.SemaphoreType.DMA,
            pltpu.SemaphoreType.DMA,
            pltpu.VMEM_SHARED((NP, 16), _f32),
        ],
    )
    return f(col3, ew2)


def _hops_body(hf, rc4, ew3, sq_hbm, dv_hbm, of,
               idxr, ewr, gb0, gb1, gb2, scb, dvb,
               is0, is1, is2, is3, is4, is5, gs0, gs1, gs2,
               acc1, acc2):
    c = lax.axis_index("c")
    s = lax.axis_index("s")
    nb = ND * s
    hoff = c * NP
    gbs = (gb0, gb1, gb2)
    gss = (gs0, gs1, gs2)
    iss = (is0, is1, is2, is3, is4, is5)

    pltpu.sync_copy(sq_hbm.at[pl.ds(nb, ND)], scb)
    pltpu.sync_copy(dv_hbm.at[pl.ds(nb, ND)], dvb)

    for j in range(ND // CB):
        pltpu.sync_copy(hf.at[pl.ds(hoff + nb + CB * j, CB)], gb0)

        @plsc.parallel_loop(0, CB, unroll=8)
        def _(e):
            bc = plsc.load_gather(dvb, [_full16(CB * j + e)])
            for k in range(FH // 16):
                gb0[e, pl.ds(16 * k, 16)] = gb0[e, pl.ds(16 * k, 16)] * bc
        pltpu.sync_copy(gb0, acc2.at[pl.ds(nb + CB * j, CB)])
        pltpu.sync_copy(gb0, acc1.at[pl.ds(nb + CB * j, CB)])
    plsc.subcore_barrier()

    def fire_idx(j, u):
        pltpu.async_copy(rc4.at[s, j], idxr.at[u], iss[u])
        pltpu.async_copy(ew3.at[s, j], ewr.at[pl.ds(u * CB, CB)], iss[u])

    def wait_idx(j, u):
        pltpu.make_async_copy(rc4.at[s, j], idxr.at[u], iss[u]).wait()
        pltpu.make_async_copy(
            ew3.at[s, j], ewr.at[pl.ds(u * CB, CB)], iss[u]).wait()

    def _hop(src, dst):
        def _work(j, u, tail):
            b = u % 3
            pltpu.make_async_copy(
                src.at[idxr.at[u, 0]], gbs[b], gss[b]).wait()

            @plsc.parallel_loop(0, CB, unroll=4)
            def _(e):
                bc = plsc.load_gather(ewr, [_full16(u * CB + e)])
                for k in range(FH // 16):
                    gbs[b][e, pl.ds(16 * k, 16)] = (
                        gbs[b][e, pl.ds(16 * k, 16)] * bc)
            pltpu.sync_copy(gbs[b], dst.at[idxr.at[u, 1]], add=True)
            if tail:
                return
            jn6 = j + 6

            @pl.when(jn6 < EC)
            def _():
                fire_idx(jn6, u)
            jn3 = j + 3
            u3 = (u + 3) % 6

            @pl.when(jn3 < EC)
            def _():
                wait_idx(jn3, u3)
                pltpu.async_copy(src.at[idxr.at[u3, 0]], gbs[b], gss[b])

        for u in range(6):
            fire_idx(u, u)
        for u in range(3):
            wait_idx(u, u)
            pltpu.async_copy(src.at[idxr.at[u, 0]], gbs[u], gss[u])

        def _round(r, _):
            for u in range(6):
                _work(6 * r + u, u, False)
            return 0
        lax.fori_loop(0, EC // 6, _round, 0)
        _work(EC - 1, 0, True)
        plsc.subcore_barrier()

    _hop(acc2, acc1)

    for j in range(ND // CB):
        pltpu.sync_copy(acc1.at[pl.ds(nb + CB * j, CB)], gb0)

        @plsc.parallel_loop(0, CB, unroll=8)
        def _(e):
            bc = plsc.load_gather(scb, [_full16(CB * j + e)])
            for k in range(FH // 16):
                gb0[e, pl.ds(16 * k, 16)] = gb0[e, pl.ds(16 * k, 16)] * bc
        pltpu.sync_copy(gb0, acc2.at[pl.ds(nb + CB * j, CB)])
        pltpu.sync_copy(gb0, acc1.at[pl.ds(nb + CB * j, CB)])
    plsc.subcore_barrier()

    _hop(acc2, acc1)

    for j in range(ND // CB):
        pltpu.sync_copy(acc1.at[pl.ds(nb + CB * j, CB)], gb0)

        @plsc.parallel_loop(0, CB, unroll=8)
        def _(e):
            bc = plsc.load_gather(dvb, [_full16(CB * j + e)])
            for k in range(FH // 16):
                gb0[e, pl.ds(16 * k, 16)] = gb0[e, pl.ds(16 * k, 16)] * bc
        pltpu.sync_copy(gb0, of.at[pl.ds(hoff + nb + CB * j, CB)])


def _hops(hf, rc4, ew3, sq_hbm, dv_hbm):
    mesh = plsc.VectorSubcoreMesh(core_axis_name="c", subcore_axis_name="s")
    f = pl.kernel(
        _hops_body,
        out_type=jax.ShapeDtypeStruct((2 * NP, FH), _f32),
        mesh=mesh,
        compiler_params=_SC_PARAMS,
        scratch_types=[
            pltpu.VMEM((6, 2, CB), _i32),
            pltpu.VMEM((6 * CB,), _f32),
            pltpu.VMEM((CB, FH), _f32),
            pltpu.VMEM((CB, FH), _f32),
            pltpu.VMEM((CB, FH), _f32),
            pltpu.VMEM((ND,), _f32),
            pltpu.VMEM((ND,), _f32),
            pltpu.SemaphoreType.DMA,
            pltpu.SemaphoreType.DMA,
            pltpu.SemaphoreType.DMA,
            pltpu.SemaphoreType.DMA,
            pltpu.SemaphoreType.DMA,
            pltpu.SemaphoreType.DMA,
            pltpu.SemaphoreType.DMA,
            pltpu.SemaphoreType.DMA,
            pltpu.SemaphoreType.DMA,
            pltpu.VMEM_SHARED((NP, FH), _f32),
            pltpu.VMEM_SHARED((NP, FH), _f32),
        ],
    )
    return f(hf, rc4, ew3, sq_hbm, dv_hbm)


def kernel(x, edge_index, edge_weight, ln_gamma, ln_beta, W, b):
    row = edge_index[0]
    col = edge_index[1]
    pad_e = EP - E
    row3 = jnp.pad(row, (0, pad_e)).reshape(NSUB, EC, CB)
    col3 = jnp.pad(col, (0, pad_e)).reshape(NSUB, EC, CB)
    rc4 = jnp.stack([row3, col3], axis=2)
    ew2 = jnp.pad(edge_weight, (0, pad_e)).reshape(NSUB, EC * CB)
    ew3 = ew2.reshape(NSUB, EC, CB)
    xp = jnp.pad(x, ((0, NP - N), (0, 0)))

    dinv, sq = _deg(col3, ew2)
    h3 = _layernorm_split(xp, ln_gamma, ln_beta)
    hf = _hops(h3.reshape(2 * NP, FH), rc4, ew3, sq, dinv)

    out = _project(hf.reshape(2, NP, FH), W, b)
    return out[:N]
